# baseline (speedup 1.0000x reference)
import functools

import jax
import jax.numpy as jnp
from jax import lax
from jax.experimental import pallas as pl
from jax.experimental.pallas import tpu as pltpu
from jax.experimental.pallas import tpu_sc as plsc

BATCH = 16384
SEQ = 20
SEQ_PAD = 24
LANES = 128
EMBED_DIM = 32
HALF = 16

NUM_CORES = 2
NUM_SUBCORES = 16
NUM_WORKERS = NUM_CORES * NUM_SUBCORES
B_PER_W = BATCH // NUM_WORKERS

CHUNK_B = 32
CHUNK_ROWS = CHUNK_B * SEQ_PAD
NUM_CHUNKS = B_PER_W // CHUNK_B


def _sc_embed_sum(table, x_pad):
    mesh = plsc.VectorSubcoreMesh(core_axis_name="c", subcore_axis_name="s")

    @functools.partial(
        pl.kernel,
        out_type=jax.ShapeDtypeStruct((BATCH, EMBED_DIM), jnp.float32),
        mesh=mesh,
        compiler_params=pltpu.CompilerParams(use_tc_tiling_on_sc=False),
        scratch_types=[
            pltpu.VMEM((B_PER_W, SEQ_PAD), jnp.int32),
            pltpu.VMEM((CHUNK_ROWS, EMBED_DIM), jnp.float32),
            pltpu.VMEM((CHUNK_ROWS, EMBED_DIM), jnp.float32),
            pltpu.VMEM((B_PER_W, EMBED_DIM), jnp.float32),
            pltpu.SemaphoreType.DMA,
            pltpu.SemaphoreType.DMA,
        ],
    )
    def k(table_hbm, idx_hbm, out_hbm, idx_v, rows0, rows1, out_v, sem0, sem1):
        wid = lax.axis_index("s") * NUM_CORES + lax.axis_index("c")
        base_b = wid * B_PER_W
        pltpu.sync_copy(idx_hbm.at[pl.ds(base_b, B_PER_W), pl.ds(0, SEQ_PAD)],
                        idx_v)

        rows = (rows0, rows1)
        sems = (sem0, sem1)

        def fire(c):
            buf, sem = rows[c % 2], sems[c % 2]
            cps = []
            for j in range(CHUNK_B):
                cps.append(pltpu.async_copy(
                    table_hbm.at[idx_v.at[c * CHUNK_B + j, :]],
                    buf.at[pl.ds(j * SEQ_PAD, SEQ_PAD)],
                    sem))
            return cps

        pending = fire(0)
        for c in range(NUM_CHUNKS):
            for cp in pending:
                cp.wait()
            if c + 1 < NUM_CHUNKS:
                pending = fire(c + 1)
            buf = rows[c % 2]

            @pl.loop(0, CHUNK_B)
            def _(b, _c=c, _buf=buf):
                r0 = b * SEQ_PAD
                acc0 = _buf[r0, pl.ds(0, HALF)]
                acc1 = _buf[r0, pl.ds(HALF, HALF)]
                for l in range(1, SEQ):
                    acc0 = acc0 + _buf[r0 + l, pl.ds(0, HALF)]
                    acc1 = acc1 + _buf[r0 + l, pl.ds(HALF, HALF)]
                ob = _c * CHUNK_B + b
                out_v[ob, pl.ds(0, HALF)] = acc0
                out_v[ob, pl.ds(HALF, HALF)] = acc1

        pltpu.sync_copy(out_v, out_hbm.at[pl.ds(base_b, B_PER_W)])

    return k(table, x_pad)


def kernel(x, action_emb):
    x_pad = jnp.pad(x.astype(jnp.int32), ((0, 0), (0, LANES - SEQ)))
    return _sc_embed_sum(action_emb, x_pad)


# speedup vs baseline: 5.3789x; 5.3789x over previous
import functools

import jax
import jax.numpy as jnp
from jax import lax
from jax.experimental import pallas as pl
from jax.experimental.pallas import tpu as pltpu
from jax.experimental.pallas import tpu_sc as plsc

BATCH = 16384
SEQ = 20
SEQ_PAD = 24
LANES = 128
EMBED_DIM = 32
HALF = 16

NUM_CORES = 2
NUM_SUBCORES = 16
NUM_WORKERS = NUM_CORES * NUM_SUBCORES
B_PER_W = BATCH // NUM_WORKERS

CHUNK_B = 16
CHUNK_ROWS = CHUNK_B * SEQ_PAD
NUM_CHUNKS = B_PER_W // CHUNK_B


def _sc_embed_sum(table, x_pad, ar):
    mesh = plsc.VectorSubcoreMesh(core_axis_name="c", subcore_axis_name="s")

    @functools.partial(
        pl.kernel,
        out_type=jax.ShapeDtypeStruct((BATCH, EMBED_DIM), jnp.float32),
        mesh=mesh,
        compiler_params=pltpu.CompilerParams(use_tc_tiling_on_sc=False),
        scratch_types=[
            pltpu.VMEM((B_PER_W,), jnp.int32),
            pltpu.VMEM((B_PER_W, LANES), jnp.int32),
            pltpu.VMEM((CHUNK_ROWS, EMBED_DIM), jnp.float32),
            pltpu.VMEM((CHUNK_ROWS, EMBED_DIM), jnp.float32),
            pltpu.VMEM((B_PER_W, EMBED_DIM), jnp.float32),
            pltpu.SemaphoreType.DMA,
            pltpu.SemaphoreType.DMA,
        ],
    )
    def k(table_hbm, idx_hbm, ar_hbm, out_hbm, iota_v, idx_v, rows0, rows1,
          out_v, sem0, sem1):
        wid = lax.axis_index("s") * NUM_CORES + lax.axis_index("c")
        base_b = wid * B_PER_W

        pltpu.sync_copy(ar_hbm.at[pl.ds(base_b, B_PER_W)], iota_v)
        pltpu.async_copy(idx_hbm.at[iota_v], idx_v, sem0).wait()

        rows = (rows0, rows1)
        sems = (sem0, sem1)

        def fire(c):
            buf, sem = rows[c % 2], sems[c % 2]
            cps = []
            for j in range(CHUNK_B):
                cps.append(pltpu.async_copy(
                    table_hbm.at[idx_v.at[c * CHUNK_B + j, pl.ds(0, SEQ_PAD)]],
                    buf.at[pl.ds(j * SEQ_PAD, SEQ_PAD)],
                    sem))
            return cps

        pending = fire(0)
        for c in range(NUM_CHUNKS):
            for cp in pending:
                cp.wait()
            if c + 1 < NUM_CHUNKS:
                pending = fire(c + 1)
            buf = rows[c % 2]

            @pl.loop(0, CHUNK_B)
            def _(b, _c=c, _buf=buf):
                r0 = b * SEQ_PAD
                acc0 = _buf[r0, pl.ds(0, HALF)]
                acc1 = _buf[r0, pl.ds(HALF, HALF)]
                for l in range(1, SEQ):
                    acc0 = acc0 + _buf[r0 + l, pl.ds(0, HALF)]
                    acc1 = acc1 + _buf[r0 + l, pl.ds(HALF, HALF)]
                ob = _c * CHUNK_B + b
                out_v[ob, pl.ds(0, HALF)] = acc0
                out_v[ob, pl.ds(HALF, HALF)] = acc1

        pltpu.sync_copy(out_v, out_hbm.at[pl.ds(base_b, B_PER_W)])

    return k(table, x_pad, ar)


def kernel(x, action_emb):
    x_pad = jnp.pad(x.astype(jnp.int32), ((0, 0), (0, LANES - SEQ)),
                    mode="edge")
    ar = jnp.arange(BATCH, dtype=jnp.int32)
    return _sc_embed_sum(action_emb, x_pad, ar)


# speedup vs baseline: 6.7050x; 1.2465x over previous
import functools

import jax
import jax.numpy as jnp
from jax import lax
from jax.experimental import pallas as pl
from jax.experimental.pallas import tpu as pltpu
from jax.experimental.pallas import tpu_sc as plsc

BATCH = 16384
SEQ = 20
EMBED_DIM = 32
HALF = 16
LANES = 128

NUM_CORES = 2
NUM_SUBCORES = 16
NUM_WORKERS = NUM_CORES * NUM_SUBCORES
B_PER_W = BATCH // NUM_WORKERS
IDX_PER_W = B_PER_W * SEQ
IDX_ROWS_W = IDX_PER_W // LANES

CHUNK_B = 64
CHUNK_IDX = CHUNK_B * SEQ
CHUNK_IDX_ROWS = CHUNK_IDX // LANES
NUM_CHUNKS = B_PER_W // CHUNK_B


def _sc_embed_sum(table, x_r):
    mesh = plsc.VectorSubcoreMesh(core_axis_name="c", subcore_axis_name="s")

    @functools.partial(
        pl.kernel,
        out_type=jax.ShapeDtypeStruct((BATCH, EMBED_DIM), jnp.float32),
        mesh=mesh,
        compiler_params=pltpu.CompilerParams(use_tc_tiling_on_sc=False),
        scratch_types=[
            pltpu.VMEM((IDX_ROWS_W, LANES), jnp.int32),
            pltpu.VMEM((CHUNK_IDX, EMBED_DIM), jnp.float32),
            pltpu.VMEM((CHUNK_IDX, EMBED_DIM), jnp.float32),
            pltpu.VMEM((B_PER_W, EMBED_DIM), jnp.float32),
            pltpu.SemaphoreType.DMA,
            pltpu.SemaphoreType.DMA,
        ],
    )
    def k(table_hbm, idx_hbm, out_hbm, idx_v, rows0, rows1, out_v, sem0, sem1):
        wid = lax.axis_index("s") * NUM_CORES + lax.axis_index("c")
        base_b = wid * B_PER_W
        pltpu.sync_copy(idx_hbm.at[pl.ds(wid * IDX_ROWS_W, IDX_ROWS_W), :],
                        idx_v)

        rows = (rows0, rows1)
        sems = (sem0, sem1)

        def fire(c):
            buf, sem = rows[c % 2], sems[c % 2]
            cps = []
            for g in range(CHUNK_IDX_ROWS):
                cps.append(pltpu.async_copy(
                    table_hbm.at[idx_v.at[c * CHUNK_IDX_ROWS + g, :]],
                    buf.at[pl.ds(g * LANES, LANES)],
                    sem))
            return cps

        pending = fire(0)
        for c in range(NUM_CHUNKS):
            for cp in pending:
                cp.wait()
            if c + 1 < NUM_CHUNKS:
                pending = fire(c + 1)
            buf = rows[c % 2]

            @pl.loop(0, CHUNK_B)
            def _(b, _c=c, _buf=buf):
                r0 = b * SEQ
                acc0 = _buf[r0, pl.ds(0, HALF)]
                acc1 = _buf[r0, pl.ds(HALF, HALF)]
                for l in range(1, SEQ):
                    acc0 = acc0 + _buf[r0 + l, pl.ds(0, HALF)]
                    acc1 = acc1 + _buf[r0 + l, pl.ds(HALF, HALF)]
                ob = _c * CHUNK_B + b
                out_v[ob, pl.ds(0, HALF)] = acc0
                out_v[ob, pl.ds(HALF, HALF)] = acc1

        pltpu.sync_copy(out_v, out_hbm.at[pl.ds(base_b, B_PER_W)])

    return k(table, x_r)


def kernel(x, action_emb):
    x_r = x.astype(jnp.int32).reshape(BATCH * SEQ // LANES, LANES)
    return _sc_embed_sum(action_emb, x_r)


# speedup vs baseline: 6.8912x; 1.0278x over previous
import functools

import jax
import jax.numpy as jnp
from jax import lax
from jax.experimental import pallas as pl
from jax.experimental.pallas import tpu as pltpu
from jax.experimental.pallas import tpu_sc as plsc

BATCH = 16384
SEQ = 20
EMBED_DIM = 32
HALF = 16
LANES = 128

NUM_CORES = 2
NUM_SUBCORES = 16
NUM_WORKERS = NUM_CORES * NUM_SUBCORES
B_PER_W = BATCH // NUM_WORKERS
IDX_PER_W = B_PER_W * SEQ

CHUNK_B = 64
CHUNK_IDX = CHUNK_B * SEQ
CHUNK_STREAMS = CHUNK_IDX // LANES
NUM_CHUNKS = B_PER_W // CHUNK_B

_MESH = plsc.VectorSubcoreMesh(core_axis_name="c", subcore_axis_name="s")


def _worker_id():
    return lax.axis_index("s") * NUM_CORES + lax.axis_index("c")


@functools.partial(
    pl.kernel,
    out_type=jax.ShapeDtypeStruct((BATCH * SEQ,), jnp.int32),
    mesh=_MESH,
    compiler_params=pltpu.CompilerParams(use_tc_tiling_on_sc=True),
    scratch_types=[
        pltpu.VMEM((B_PER_W, SEQ), jnp.int32),
        pltpu.VMEM((IDX_PER_W,), jnp.int32),
    ],
)
def _sc_flatten(x_hbm, out_hbm, xin_v, xout_v):
    wid = _worker_id()
    pltpu.sync_copy(x_hbm.at[pl.ds(wid * B_PER_W, B_PER_W), :], xin_v)

    @pl.loop(0, B_PER_W)
    def _(r):
        a = xin_v[r, pl.ds(0, HALF)]
        b = xin_v[r, pl.ds(SEQ - HALF, HALF)]
        xout_v[pl.ds(r * SEQ, HALF)] = a
        xout_v[pl.ds(r * SEQ + (SEQ - HALF), HALF)] = b

    pltpu.sync_copy(xout_v, out_hbm.at[pl.ds(wid * IDX_PER_W, IDX_PER_W)])


@functools.partial(
    pl.kernel,
    out_type=jax.ShapeDtypeStruct((BATCH, EMBED_DIM), jnp.float32),
    mesh=_MESH,
    compiler_params=pltpu.CompilerParams(use_tc_tiling_on_sc=False),
    scratch_types=[
        pltpu.VMEM((IDX_PER_W,), jnp.int32),
        pltpu.VMEM((CHUNK_IDX, EMBED_DIM), jnp.float32),
        pltpu.VMEM((CHUNK_IDX, EMBED_DIM), jnp.float32),
        pltpu.VMEM((B_PER_W, EMBED_DIM), jnp.float32),
        pltpu.SemaphoreType.DMA,
        pltpu.SemaphoreType.DMA,
    ],
)
def _sc_embed_sum(table_hbm, idx_hbm, out_hbm, idx_v, rows0, rows1, out_v,
                  sem0, sem1):
    wid = _worker_id()
    base_b = wid * B_PER_W
    pltpu.sync_copy(idx_hbm.at[pl.ds(wid * IDX_PER_W, IDX_PER_W)], idx_v)

    rows = (rows0, rows1)
    sems = (sem0, sem1)

    def fire(c):
        buf, sem = rows[c % 2], sems[c % 2]
        cps = []
        for g in range(CHUNK_STREAMS):
            cps.append(pltpu.async_copy(
                table_hbm.at[idx_v.at[pl.ds(c * CHUNK_IDX + g * LANES, LANES)]],
                buf.at[pl.ds(g * LANES, LANES)],
                sem))
        return cps

    pending = fire(0)
    for c in range(NUM_CHUNKS):
        for cp in pending:
            cp.wait()
        if c + 1 < NUM_CHUNKS:
            pending = fire(c + 1)
        buf = rows[c % 2]

        @pl.loop(0, CHUNK_B)
        def _(b, _c=c, _buf=buf):
            r0 = b * SEQ
            acc0 = _buf[r0, pl.ds(0, HALF)]
            acc1 = _buf[r0, pl.ds(HALF, HALF)]
            for l in range(1, SEQ):
                acc0 = acc0 + _buf[r0 + l, pl.ds(0, HALF)]
                acc1 = acc1 + _buf[r0 + l, pl.ds(HALF, HALF)]
            ob = _c * CHUNK_B + b
            out_v[ob, pl.ds(0, HALF)] = acc0
            out_v[ob, pl.ds(HALF, HALF)] = acc1

    pltpu.sync_copy(out_v, out_hbm.at[pl.ds(base_b, B_PER_W)])


def kernel(x, action_emb):
    x_flat = _sc_flatten(x.astype(jnp.int32))
    return _sc_embed_sum(action_emb, x_flat)
